# Initial kernel scaffold; baseline (speedup 1.0000x reference)
import jax
import jax.numpy as jnp
from jax.experimental import pallas as pl


def kernel(pred, target, mask):
    raise NotImplementedError("write your pallas kernel here")


# speedup vs baseline: 1.1734x; 1.1734x over previous
import jax
import jax.numpy as jnp
from jax.experimental import pallas as pl
from jax.experimental.pallas import tpu as pltpu

_ROWS = 4 * 2048
_COLS = 4096
_BLOCK_ROWS = 256
_GRID = _ROWS // _BLOCK_ROWS


def _mse_kernel(p_ref, t_ref, m_ref, out_ref, acc_ref):
    i = pl.program_id(0)

    @pl.when(i == 0)
    def _init():
        acc_ref[0] = 0.0
        acc_ref[1] = 0.0

    mf = (m_ref[...] == 1).astype(jnp.float32)
    d = (p_ref[...] - t_ref[...]) * mf
    acc_ref[0] += jnp.sum(d * d)
    acc_ref[1] += jnp.sum(mf)

    @pl.when(i == _GRID - 1)
    def _fini():
        out_ref[0] = acc_ref[0] / acc_ref[1]


def kernel(pred, target, mask):
    p2 = pred.reshape(_ROWS, _COLS)
    t2 = target.reshape(_ROWS, _COLS)
    m2 = mask.astype(jnp.int32).reshape(_ROWS, _COLS)

    in_spec = pl.BlockSpec((_BLOCK_ROWS, _COLS), lambda i: (i, 0))
    out = pl.pallas_call(
        _mse_kernel,
        grid=(_GRID,),
        in_specs=[in_spec, in_spec, in_spec],
        out_specs=pl.BlockSpec(memory_space=pltpu.SMEM),
        out_shape=jax.ShapeDtypeStruct((1,), jnp.float32),
        scratch_shapes=[pltpu.SMEM((2,), jnp.float32)],
    )(p2, t2, m2)
    return out[0]
